# baseline (speedup 1.0000x reference)
import functools

import jax
import jax.numpy as jnp
from jax import lax
from jax.experimental import pallas as pl
from jax.experimental.pallas import tpu as pltpu
from jax.experimental.pallas import tpu_sc as plsc

B, L, D = 8, 2048, 768
C, K = 512, 4

NC, NS, LANES = 2, 16, 16
NW = NC * NS
HB = B // 2
NCH_H = HB * C
CPW = NCH_H // NW
WPB = NW // HB
BLK = 16
NBLK = CPW // BLK
NH = NBLK // 2

_mesh = plsc.VectorSubcoreMesh(core_axis_name="c", subcore_axis_name="s")


def _make_sc_pool(b_base):
    @functools.partial(
        pl.kernel,
        mesh=_mesh,
        compiler_params=pltpu.CompilerParams(needs_layout_passes=False),
        out_type=(
            jax.ShapeDtypeStruct((NCH_H, D), jnp.float32),
            jax.ShapeDtypeStruct((HB, C), jnp.int32),
            jax.ShapeDtypeStruct((HB, C), jnp.int32),
            jax.ShapeDtypeStruct((HB, C), jnp.int32),
        ),
        scratch_types=[
            pltpu.VMEM((CPW * K,), jnp.int32),
            pltpu.VMEM((BLK * K,), jnp.int32),
            pltpu.VMEM((BLK * K,), jnp.int32),
            pltpu.VMEM((BLK * K, D), jnp.float32),
            pltpu.VMEM((BLK * K, D), jnp.float32),
            pltpu.VMEM((BLK, D), jnp.float32),
            pltpu.VMEM((BLK, D), jnp.float32),
            pltpu.VMEM((L,), jnp.int32),
            pltpu.VMEM((L,), jnp.int32),
            pltpu.VMEM((L,), jnp.int32),
            pltpu.VMEM((CPW,), jnp.int32),
            pltpu.VMEM((CPW,), jnp.int32),
            pltpu.VMEM((CPW,), jnp.int32),
            pltpu.SemaphoreType.DMA,
            pltpu.SemaphoreType.DMA,
            pltpu.SemaphoreType.DMA,
            pltpu.SemaphoreType.DMA,
            pltpu.SemaphoreType.DMA,
        ],
    )
    def _sc_pool_h(tens_hbm, idx_hbm, pad_hbm, reg_hbm, sp_hbm,
                   pooled_hbm, mp_hbm, mr_hbm, ms_hbm,
                   idxr_v, idxb0_v, idxb1_v, rows0_v, rows1_v,
                   pool0_v, pool1_v, pad_v, reg_v, sp_v, mpb_v, mrb_v, msb_v,
                   semg0, semg1, sems0, sems1, semm):
        cid = lax.axis_index("c")
        sid = lax.axis_index("s")
        wid = sid * NC + cid
        b_loc = wid // WPB
        b = b_base + b_loc
        c0 = (wid % WPB) * CPW
        chunk0 = wid * CPW

        pltpu.sync_copy(idx_hbm.at[b, pl.ds(c0 * K, CPW * K)], idxr_v)
        pltpu.async_copy(pad_hbm.at[b], pad_v, semm)
        pltpu.async_copy(reg_hbm.at[b], reg_v, semm)
        pltpu.async_copy(sp_hbm.at[b], sp_v, semm)

        off = b * L

        def _fill_idx(idxb_v, g):
            for j in range((BLK * K) // LANES):
                idxb_v[pl.ds(j * LANES, LANES)] = (
                    idxr_v[pl.ds(g * BLK * K + j * LANES, LANES)] + off)

        def _start_gather(idxb_v, rows_v, sem):
            pltpu.async_copy(tens_hbm.at[idxb_v], rows_v, sem)

        def _wait_gather(idxb_v, rows_v, sem):
            pltpu.make_async_copy(tens_hbm.at[idxb_v], rows_v, sem).wait()

        def _compute(rows_v, pool_v):
            def _col_body(s, inner):
                base = s * LANES
                for c in range(BLK):
                    r0 = rows_v[c * K + 0, pl.ds(base, LANES)]
                    r1 = rows_v[c * K + 1, pl.ds(base, LANES)]
                    r2 = rows_v[c * K + 2, pl.ds(base, LANES)]
                    r3 = rows_v[c * K + 3, pl.ds(base, LANES)]
                    pool_v[c, pl.ds(base, LANES)] = (r0 + r1 + r2 + r3) * 0.25
                return inner

            lax.fori_loop(0, D // LANES, _col_body, 0)

        def _wait_store(pool_v, sem):
            pltpu.make_async_copy(pool_v, pooled_hbm.at[pl.ds(chunk0, BLK)],
                                  sem).wait()

        _fill_idx(idxb0_v, 0)
        _start_gather(idxb0_v, rows0_v, semg0)
        _fill_idx(idxb1_v, 1)
        _start_gather(idxb1_v, rows1_v, semg1)

        def _half(h, idxb_v, rows_v, pool_v, semg, sems, g, gnext):
            _wait_gather(idxb_v, rows_v, semg)

            @pl.when(h > 0)
            def _():
                _wait_store(pool_v, sems)

            _compute(rows_v, pool_v)

            @pl.when(gnext < NBLK)
            def _():
                _fill_idx(idxb_v, gnext)
                _start_gather(idxb_v, rows_v, semg)

            pltpu.async_copy(pool_v,
                             pooled_hbm.at[pl.ds(chunk0 + g * BLK, BLK)],
                             sems)

        def _blk2_body(h, carry):
            _half(h, idxb0_v, rows0_v, pool0_v, semg0, sems0,
                  2 * h, 2 * h + 2)
            _half(h, idxb1_v, rows1_v, pool1_v, semg1, sems1,
                  2 * h + 1, 2 * h + 3)
            return carry

        lax.fori_loop(0, NH, _blk2_body, 0)
        _wait_store(pool0_v, sems0)
        _wait_store(pool1_v, sems1)

        pltpu.make_async_copy(pad_hbm.at[b], pad_v, semm).wait()
        pltpu.make_async_copy(reg_hbm.at[b], reg_v, semm).wait()
        pltpu.make_async_copy(sp_hbm.at[b], sp_v, semm).wait()
        lane = lax.iota(jnp.int32, LANES)

        def _msk_body(g, carry):
            cidx = g * LANES + lane
            psum = jnp.zeros((LANES,), jnp.int32)
            rsum = jnp.zeros((LANES,), jnp.int32)
            sprod = jnp.ones((LANES,), jnp.int32)
            for k in range(K):
                tok = plsc.load_gather(idxr_v, [cidx * K + k])
                psum = psum + plsc.load_gather(pad_v, [tok])
                rsum = rsum + plsc.load_gather(reg_v, [tok])
                sprod = sprod * plsc.load_gather(sp_v, [tok])
            mp = (psum != 0).astype(jnp.int32)
            mr = (rsum != 0).astype(jnp.int32)
            ms = (sprod != 0).astype(jnp.int32)
            ms = jnp.where(mp == 0, -1, ms)
            mpb_v[pl.ds(g * LANES, LANES)] = mp
            mrb_v[pl.ds(g * LANES, LANES)] = mr
            msb_v[pl.ds(g * LANES, LANES)] = ms
            return carry

        lax.fori_loop(0, CPW // LANES, _msk_body, 0)
        pltpu.sync_copy(mpb_v, mp_hbm.at[b_loc, pl.ds(c0, CPW)])
        pltpu.sync_copy(mrb_v, mr_hbm.at[b_loc, pl.ds(c0, CPW)])
        pltpu.sync_copy(msb_v, ms_hbm.at[b_loc, pl.ds(c0, CPW)])

    return _sc_pool_h


_sc_pool_a = _make_sc_pool(0)
_sc_pool_b = _make_sc_pool(HB)


def _tc_body(x_ref, w_ref, bias_ref, out_ref):
    acc = jnp.dot(x_ref[...], w_ref[...], preferred_element_type=jnp.float32)
    out_ref[...] = jnp.tanh(acc + bias_ref[...])


def _tc_body_cr(x_ref, w_ref, bias_ref, regc_ref, regm_ref, out_ref, cr_ref):
    acc = jnp.dot(x_ref[...], w_ref[...], preferred_element_type=jnp.float32)
    out_ref[...] = jnp.tanh(acc + bias_ref[...])
    num = regc_ref[...].sum().astype(jnp.float32)
    den = regm_ref[...].sum().astype(jnp.float32)
    cr_ref[0, 0] = num / den


_tc_proj = pl.pallas_call(
    _tc_body,
    grid=(1,),
    in_specs=[
        pl.BlockSpec((NCH_H, D), lambda i: (0, 0)),
        pl.BlockSpec((D, D), lambda i: (0, 0)),
        pl.BlockSpec((1, D), lambda i: (0, 0)),
    ],
    out_specs=pl.BlockSpec((NCH_H, D), lambda i: (0, 0)),
    out_shape=jax.ShapeDtypeStruct((NCH_H, D), jnp.float32),
)

_tc_proj_cr = pl.pallas_call(
    _tc_body_cr,
    grid=(1,),
    in_specs=[
        pl.BlockSpec((NCH_H, D), lambda i: (0, 0)),
        pl.BlockSpec((D, D), lambda i: (0, 0)),
        pl.BlockSpec((1, D), lambda i: (0, 0)),
        pl.BlockSpec((B, C), lambda i: (0, 0)),
        pl.BlockSpec((B, L), lambda i: (0, 0)),
    ],
    out_specs=[
        pl.BlockSpec((NCH_H, D), lambda i: (0, 0)),
        pl.BlockSpec(memory_space=pltpu.SMEM),
    ],
    out_shape=[
        jax.ShapeDtypeStruct((NCH_H, D), jnp.float32),
        jax.ShapeDtypeStruct((1, 1), jnp.float32),
    ],
)


def kernel(tensors_batch, indices_batch, padding_mask, regular_tokens_mask,
           seq_pair_mask, W, b):
    assert tensors_batch.shape == (B, L, D)
    assert indices_batch.shape == (B, C, K)

    tens_flat = tensors_batch.reshape(B * L, D)
    idx_flat = indices_batch.reshape(B, C * K)
    bias = b.reshape(1, D)

    pooled_a, mp_a, mr_a, ms_a = _sc_pool_a(
        tens_flat, idx_flat, padding_mask, regular_tokens_mask, seq_pair_mask)
    pooled_b, mp_b, mr_b, ms_b = _sc_pool_b(
        tens_flat, idx_flat, padding_mask, regular_tokens_mask, seq_pair_mask)

    mp = jnp.concatenate([mp_a, mp_b], axis=0)
    mr = jnp.concatenate([mr_a, mr_b], axis=0)
    ms = jnp.concatenate([ms_a, ms_b], axis=0)

    joint_a = _tc_proj(pooled_a, W, bias)
    joint_b, cr = _tc_proj_cr(pooled_b, W, bias, mr, regular_tokens_mask)

    joint = jnp.concatenate([joint_a, joint_b], axis=0)
    return (joint.reshape(B, C, D), mp, mr, ms, cr[0, 0])


# speedup vs baseline: 1.4690x; 1.4690x over previous
import functools

import jax
import jax.numpy as jnp
from jax import lax
from jax.experimental import pallas as pl
from jax.experimental.pallas import tpu as pltpu
from jax.experimental.pallas import tpu_sc as plsc

B, L, D = 8, 2048, 768
C, K = 512, 4

NC, NS, LANES = 2, 16, 16
NW = NC * NS
NCHUNKS = B * C
CPW = NCHUNKS // NW
WPB = NW // B
BLK = 16
NBLK = CPW // BLK
NH = NBLK // 2

_mesh = plsc.VectorSubcoreMesh(core_axis_name="c", subcore_axis_name="s")


@functools.partial(
    pl.kernel,
    mesh=_mesh,
    compiler_params=pltpu.CompilerParams(needs_layout_passes=False),
    out_type=(
        jax.ShapeDtypeStruct((NCHUNKS, D), jnp.float32),
        jax.ShapeDtypeStruct((B, C), jnp.int32),
        jax.ShapeDtypeStruct((B, C), jnp.int32),
        jax.ShapeDtypeStruct((B, C), jnp.int32),
    ),
    scratch_types=[
        pltpu.VMEM((CPW * K,), jnp.int32),
        pltpu.VMEM((BLK * K,), jnp.int32),
        pltpu.VMEM((BLK * K,), jnp.int32),
        pltpu.VMEM((BLK * K, D), jnp.float32),
        pltpu.VMEM((BLK * K, D), jnp.float32),
        pltpu.VMEM((BLK, D), jnp.float32),
        pltpu.VMEM((BLK, D), jnp.float32),
        pltpu.VMEM((L,), jnp.int32),
        pltpu.VMEM((L,), jnp.int32),
        pltpu.VMEM((L,), jnp.int32),
        pltpu.VMEM((CPW,), jnp.int32),
        pltpu.VMEM((CPW,), jnp.int32),
        pltpu.VMEM((CPW,), jnp.int32),
        pltpu.SemaphoreType.DMA,
        pltpu.SemaphoreType.DMA,
        pltpu.SemaphoreType.DMA,
        pltpu.SemaphoreType.DMA,
        pltpu.SemaphoreType.DMA,
    ],
)
def _sc_pool(tens_hbm, idx_hbm, pad_hbm, reg_hbm, sp_hbm,
             pooled_hbm, mp_hbm, mr_hbm, ms_hbm,
             idxr_v, idxb0_v, idxb1_v, rows0_v, rows1_v,
             pool0_v, pool1_v, pad_v, reg_v, sp_v, mpb_v, mrb_v, msb_v,
             semg0, semg1, sems0, sems1, semm):
    cid = lax.axis_index("c")
    sid = lax.axis_index("s")
    wid = sid * NC + cid
    b = wid // WPB
    c0 = (wid % WPB) * CPW
    chunk0 = wid * CPW

    pltpu.sync_copy(idx_hbm.at[b, pl.ds(c0 * K, CPW * K)], idxr_v)
    pltpu.async_copy(pad_hbm.at[b], pad_v, semm)
    pltpu.async_copy(reg_hbm.at[b], reg_v, semm)
    pltpu.async_copy(sp_hbm.at[b], sp_v, semm)

    off = b * L

    def _fill_idx(idxb_v, g):
        for j in range((BLK * K) // LANES):
            idxb_v[pl.ds(j * LANES, LANES)] = (
                idxr_v[pl.ds(g * BLK * K + j * LANES, LANES)] + off)

    def _start_gather(idxb_v, rows_v, sem):
        pltpu.async_copy(tens_hbm.at[idxb_v], rows_v, sem)

    def _wait_gather(idxb_v, rows_v, sem):
        pltpu.make_async_copy(tens_hbm.at[idxb_v], rows_v, sem).wait()

    def _compute(rows_v, pool_v):
        def _col_body(s, inner):
            base = s * LANES
            for c in range(BLK):
                r0 = rows_v[c * K + 0, pl.ds(base, LANES)]
                r1 = rows_v[c * K + 1, pl.ds(base, LANES)]
                r2 = rows_v[c * K + 2, pl.ds(base, LANES)]
                r3 = rows_v[c * K + 3, pl.ds(base, LANES)]
                pool_v[c, pl.ds(base, LANES)] = (r0 + r1 + r2 + r3) * 0.25
            return inner

        lax.fori_loop(0, D // LANES, _col_body, 0)

    def _wait_store(pool_v, sem):
        pltpu.make_async_copy(pool_v, pooled_hbm.at[pl.ds(chunk0, BLK)],
                              sem).wait()

    _fill_idx(idxb0_v, 0)
    _start_gather(idxb0_v, rows0_v, semg0)
    _fill_idx(idxb1_v, 1)
    _start_gather(idxb1_v, rows1_v, semg1)

    pltpu.make_async_copy(pad_hbm.at[b], pad_v, semm).wait()
    pltpu.make_async_copy(reg_hbm.at[b], reg_v, semm).wait()
    pltpu.make_async_copy(sp_hbm.at[b], sp_v, semm).wait()
    lane = lax.iota(jnp.int32, LANES)

    def _msk_body(g, carry):
        cidx = g * LANES + lane
        psum = jnp.zeros((LANES,), jnp.int32)
        rsum = jnp.zeros((LANES,), jnp.int32)
        sprod = jnp.ones((LANES,), jnp.int32)
        for k in range(K):
            tok = plsc.load_gather(idxr_v, [cidx * K + k])
            psum = psum + plsc.load_gather(pad_v, [tok])
            rsum = rsum + plsc.load_gather(reg_v, [tok])
            sprod = sprod * plsc.load_gather(sp_v, [tok])
        mp = (psum != 0).astype(jnp.int32)
        mr = (rsum != 0).astype(jnp.int32)
        ms = (sprod != 0).astype(jnp.int32)
        ms = jnp.where(mp == 0, -1, ms)
        mpb_v[pl.ds(g * LANES, LANES)] = mp
        mrb_v[pl.ds(g * LANES, LANES)] = mr
        msb_v[pl.ds(g * LANES, LANES)] = ms
        return carry

    lax.fori_loop(0, CPW // LANES, _msk_body, 0)
    pltpu.async_copy(mpb_v, mp_hbm.at[b, pl.ds(c0, CPW)], semm)
    pltpu.async_copy(mrb_v, mr_hbm.at[b, pl.ds(c0, CPW)], semm)
    pltpu.async_copy(msb_v, ms_hbm.at[b, pl.ds(c0, CPW)], semm)

    def _half(h, idxb_v, rows_v, pool_v, semg, sems, g, gnext):
        _wait_gather(idxb_v, rows_v, semg)

        @pl.when(h > 0)
        def _():
            _wait_store(pool_v, sems)

        _compute(rows_v, pool_v)

        @pl.when(gnext < NBLK)
        def _():
            _fill_idx(idxb_v, gnext)
            _start_gather(idxb_v, rows_v, semg)

        pltpu.async_copy(pool_v, pooled_hbm.at[pl.ds(chunk0 + g * BLK, BLK)],
                         sems)

    def _blk2_body(h, carry):
        _half(h, idxb0_v, rows0_v, pool0_v, semg0, sems0, 2 * h, 2 * h + 2)
        _half(h, idxb1_v, rows1_v, pool1_v, semg1, sems1, 2 * h + 1, 2 * h + 3)
        return carry

    lax.fori_loop(0, NH, _blk2_body, 0)
    _wait_store(pool0_v, sems0)
    _wait_store(pool1_v, sems1)

    pltpu.make_async_copy(mpb_v, mp_hbm.at[b, pl.ds(c0, CPW)], semm).wait()
    pltpu.make_async_copy(mrb_v, mr_hbm.at[b, pl.ds(c0, CPW)], semm).wait()
    pltpu.make_async_copy(msb_v, ms_hbm.at[b, pl.ds(c0, CPW)], semm).wait()


RB = 2048


def _tc_body(x_ref, w_ref, bias_ref, regc_ref, regm_ref, out_ref, cr_ref):
    acc = jnp.dot(x_ref[...], w_ref[...], preferred_element_type=jnp.float32)
    out_ref[...] = jnp.tanh(acc + bias_ref[...])

    @pl.when(pl.program_id(0) == 0)
    def _():
        num = regc_ref[...].sum().astype(jnp.float32)
        den = regm_ref[...].sum().astype(jnp.float32)
        cr_ref[0, 0] = num / den


_tc_proj = pl.pallas_call(
    _tc_body,
    grid=(NCHUNKS // RB,),
    in_specs=[
        pl.BlockSpec((RB, D), lambda i: (i, 0)),
        pl.BlockSpec((D, D), lambda i: (0, 0)),
        pl.BlockSpec((1, D), lambda i: (0, 0)),
        pl.BlockSpec((B, C), lambda i: (0, 0)),
        pl.BlockSpec((B, L), lambda i: (0, 0)),
    ],
    out_specs=[
        pl.BlockSpec((RB, D), lambda i: (i, 0)),
        pl.BlockSpec(memory_space=pltpu.SMEM),
    ],
    out_shape=[
        jax.ShapeDtypeStruct((NCHUNKS, D), jnp.float32),
        jax.ShapeDtypeStruct((1, 1), jnp.float32),
    ],
)


def kernel(tensors_batch, indices_batch, padding_mask, regular_tokens_mask,
           seq_pair_mask, W, b):
    assert tensors_batch.shape == (B, L, D)
    assert indices_batch.shape == (B, C, K)

    tens_flat = tensors_batch.reshape(B * L, D)
    idx_flat = indices_batch.reshape(B, C * K)

    pooled, mp, mr, ms = _sc_pool(tens_flat, idx_flat, padding_mask,
                                  regular_tokens_mask, seq_pair_mask)
    joint, cr = _tc_proj(pooled, W, b.reshape(1, D), mr, regular_tokens_mask)
    return (joint.reshape(B, C, D), mp, mr, ms, cr[0, 0])
